# Initial kernel scaffold; baseline (speedup 1.0000x reference)
import jax
import jax.numpy as jnp
from jax.experimental import pallas as pl


def kernel(x, edge_index, W1, b1, W2, b2):
    raise NotImplementedError("write your pallas kernel here")


# speedup vs baseline: 15.9148x; 15.9148x over previous
import functools

import jax
import jax.numpy as jnp
from jax import lax
from jax.experimental import pallas as pl
from jax.experimental.pallas import tpu as pltpu
from jax.experimental.pallas import tpu_sc as plsc

_NC = 2
_NS = 16
_W = _NC * _NS
_K = 128
_R = 512


def _deg_kernel(npad, c_chunks, w=128):
  rpt = npad // _NS
  mesh = plsc.VectorSubcoreMesh(core_axis_name="c", subcore_axis_name="s")

  def body(dstw, zeros16, ones16, out_hbm, didx, ones_v, acc):
    cid = lax.axis_index("c")
    sid = lax.axis_index("s")
    wid = sid * _NC + cid
    pltpu.sync_copy(zeros16.at[pl.ds(sid * rpt, rpt)],
                    acc.at[pl.ds(sid * rpt, rpt)])
    pltpu.sync_copy(ones16, ones_v)
    plsc.subcore_barrier()

    def step(c, carry):
      pltpu.sync_copy(dstw.at[wid, c], didx)
      pltpu.sync_copy(ones_v, acc.at[didx], add=True)
      return carry

    lax.fori_loop(0, c_chunks, step, 0)
    plsc.subcore_barrier()
    pltpu.sync_copy(acc.at[pl.ds(sid * rpt, rpt)],
                    out_hbm.at[cid, pl.ds(sid * rpt, rpt)])

  return pl.kernel(
      body,
      out_type=jax.ShapeDtypeStruct((_NC, npad, w), jnp.float32),
      mesh=mesh,
      scratch_types=[
          pltpu.VMEM((_K,), jnp.int32),
          pltpu.VMEM((_K, w), jnp.float32),
          pltpu.VMEM_SHARED((npad, w), jnp.float32),
      ],
  )


def _agg_kernel(npad, c_chunks, d):
  rpt = npad // _NS
  mesh = plsc.VectorSubcoreMesh(core_axis_name="c", subcore_axis_name="s")

  def body(rows_hbm, srcw, dstw, zeros_hbm, out_hbm, sidx, didx, rows_v,
           acc, sem):
    cid = lax.axis_index("c")
    sid = lax.axis_index("s")
    wid = sid * _NC + cid
    pltpu.sync_copy(zeros_hbm.at[pl.ds(sid * rpt, rpt)],
                    acc.at[pl.ds(sid * rpt, rpt)])
    plsc.subcore_barrier()

    def step(c, carry):
      pltpu.sync_copy(srcw.at[wid, c], sidx)
      pltpu.sync_copy(dstw.at[wid, c], didx)
      pltpu.async_copy(rows_hbm.at[sidx], rows_v, sem).wait()
      pltpu.sync_copy(rows_v, acc.at[didx], add=True)
      return carry

    lax.fori_loop(0, c_chunks, step, 0)
    plsc.subcore_barrier()
    pltpu.sync_copy(acc.at[pl.ds(sid * rpt, rpt)],
                    out_hbm.at[cid, pl.ds(sid * rpt, rpt)])

  return pl.kernel(
      body,
      out_type=jax.ShapeDtypeStruct((_NC, npad, d), jnp.float32),
      mesh=mesh,
      scratch_types=[
          pltpu.VMEM((_K,), jnp.int32),
          pltpu.VMEM((_K,), jnp.int32),
          pltpu.VMEM((_K, d), jnp.float32),
          pltpu.VMEM_SHARED((npad, d), jnp.float32),
          pltpu.SemaphoreType.DMA,
      ],
  )


def _dinv(degp0, degp1):
  return lax.rsqrt(degp0[:, 0:1] + degp1[:, 0:1] + 1.0)


def _scale_body(degp_ref, x_ref, y_ref):
  y_ref[...] = x_ref[...] * _dinv(degp_ref[0], degp_ref[1])


def _dense_body(degp_ref, aggp_ref, y_ref, w1_ref, b1_ref, w2_ref, y2_ref):
  dinv = _dinv(degp_ref[0], degp_ref[1])
  pre = (aggp_ref[0] + aggp_ref[1] + y_ref[...]) * dinv
  h = jnp.dot(pre, w1_ref[...], preferred_element_type=jnp.float32)
  x1 = jnp.maximum(h + b1_ref[...], 0.0)
  h2 = jnp.dot(x1, w2_ref[...], preferred_element_type=jnp.float32)
  y2_ref[...] = h2 * dinv


def _final_body(degp_ref, aggp_ref, y2_ref, b2_ref, out_ref):
  dinv = _dinv(degp_ref[0], degp_ref[1])
  o = (aggp_ref[0] + aggp_ref[1] + y2_ref[...]) * dinv + b2_ref[...]
  m = jnp.max(o, axis=1, keepdims=True)
  s = jnp.sum(jnp.exp(o - m), axis=1, keepdims=True)
  out_ref[...] = (o - m) - jnp.log(s)


def _row_specs(npad, d):
  degp = pl.BlockSpec((2, _R, 128), lambda i: (0, i, 0))
  rows = pl.BlockSpec((_R, d), lambda i: (i, 0))
  aggp = pl.BlockSpec((2, _R, d), lambda i: (0, i, 0))
  return degp, rows, aggp


def kernel(x, edge_index, W1, b1, W2, b2):
  n, d_in = x.shape
  e = edge_index.shape[1]
  hid = W1.shape[1]
  d_out = W2.shape[1]

  npad = ((n + 1 + 255) // 256) * 256
  c_chunks = -(-e // (_W * _K))
  epad = _W * _K * c_chunks
  pad = epad - e

  x = x.astype(jnp.float32)
  src = jnp.concatenate([edge_index[0],
                         jnp.zeros((pad,), edge_index.dtype)])
  sink = n + (jnp.arange(pad, dtype=edge_index.dtype) % (npad - n))
  dst = jnp.concatenate([edge_index[1], sink])
  srcw = src.reshape(_W, c_chunks, _K)
  dstw = dst.reshape(_W, c_chunks, _K)
  x_p = jnp.pad(x, ((0, npad - n), (0, 0)))
  zeros_d = jnp.zeros((npad, d_in), jnp.float32)
  ones_d = jnp.ones((_K, d_in), jnp.float32)

  degp = _deg_kernel(npad, c_chunks, d_in)(dstw, zeros_d, ones_d)

  grid = (npad // _R,)
  degp_s, row_s, aggp_s = _row_specs(npad, d_in)

  y = pl.pallas_call(
      _scale_body,
      grid=grid,
      in_specs=[degp_s, row_s],
      out_specs=row_s,
      out_shape=jax.ShapeDtypeStruct((npad, d_in), jnp.float32),
  )(degp, x_p)

  aggp1 = _agg_kernel(npad, c_chunks, d_in)(y, srcw, dstw, zeros_d)

  y2 = pl.pallas_call(
      _dense_body,
      grid=grid,
      in_specs=[
          degp_s, aggp_s, row_s,
          pl.BlockSpec((d_in, hid), lambda i: (0, 0)),
          pl.BlockSpec((1, hid), lambda i: (0, 0)),
          pl.BlockSpec((hid, d_out), lambda i: (0, 0)),
      ],
      out_specs=pl.BlockSpec((_R, d_out), lambda i: (i, 0)),
      out_shape=jax.ShapeDtypeStruct((npad, d_out), jnp.float32),
  )(degp, aggp1, y, W1, b1.reshape(1, hid), W2)

  aggp2 = _agg_kernel(npad, c_chunks, d_out)(y2, srcw, dstw, zeros_d)

  degp_s2, row_s2, aggp_s2 = _row_specs(npad, d_out)
  out = pl.pallas_call(
      _final_body,
      grid=grid,
      in_specs=[degp_s2, aggp_s2, row_s2,
                pl.BlockSpec((1, d_out), lambda i: (0, 0))],
      out_specs=row_s2,
      out_shape=jax.ShapeDtypeStruct((npad, d_out), jnp.float32),
  )(degp, aggp2, y2, b2.reshape(1, d_out))

  return out[:n]


# speedup vs baseline: 18.0191x; 1.1322x over previous
import functools

import jax
import jax.numpy as jnp
from jax import lax
from jax.experimental import pallas as pl
from jax.experimental.pallas import tpu as pltpu
from jax.experimental.pallas import tpu_sc as plsc

_NC = 2
_NS = 16
_W = _NC * _NS
_K = 128
_R = 512


def _deg_kernel(npad, c_chunks, w=128):
  rpt = npad // _NS
  mesh = plsc.VectorSubcoreMesh(core_axis_name="c", subcore_axis_name="s")

  nb = 2
  assert c_chunks % nb == 0

  def body(dstw, zeros_hbm, ones_hbm, out_hbm, didx_all, ones_v, acc, *sems):
    cid = lax.axis_index("c")
    sid = lax.axis_index("s")
    wid = sid * _NC + cid
    pltpu.sync_copy(zeros_hbm.at[pl.ds(sid * rpt, rpt)],
                    acc.at[pl.ds(sid * rpt, rpt)])
    pltpu.sync_copy(ones_hbm, ones_v)
    pltpu.sync_copy(dstw.at[wid], didx_all)
    plsc.subcore_barrier()

    def group(j, carry):
      base = j * nb
      ds = [pltpu.async_copy(ones_v, acc.at[didx_all.at[base + b]], sems[b],
                             add=True) for b in range(nb)]
      for d_ in ds:
        d_.wait()
      return carry

    lax.fori_loop(0, c_chunks // nb, group, 0)
    plsc.subcore_barrier()
    pltpu.sync_copy(acc.at[pl.ds(sid * rpt, rpt)],
                    out_hbm.at[cid, pl.ds(sid * rpt, rpt)])

  return pl.kernel(
      body,
      out_type=jax.ShapeDtypeStruct((_NC, npad, w), jnp.float32),
      mesh=mesh,
      scratch_types=[
          pltpu.VMEM((c_chunks, _K), jnp.int32),
          pltpu.VMEM((_K, w), jnp.float32),
          pltpu.VMEM_SHARED((npad, w), jnp.float32),
      ] + [pltpu.SemaphoreType.DMA] * nb,
  )


def _agg_kernel(npad, c_chunks, d):
  rpt = npad // _NS
  mesh = plsc.VectorSubcoreMesh(core_axis_name="c", subcore_axis_name="s")

  nb = 2
  assert c_chunks % nb == 0

  def body(rows_hbm, srcw, dstw, zeros_hbm, out_hbm, sidx_all, didx_all,
           *bufs_and_sems):
    rows = bufs_and_sems[:nb]
    acc = bufs_and_sems[nb]
    gsem = bufs_and_sems[nb + 1:2 * nb + 1]
    ssem = bufs_and_sems[2 * nb + 1:]
    cid = lax.axis_index("c")
    sid = lax.axis_index("s")
    wid = sid * _NC + cid
    pltpu.sync_copy(zeros_hbm.at[pl.ds(sid * rpt, rpt)],
                    acc.at[pl.ds(sid * rpt, rpt)])
    pltpu.sync_copy(srcw.at[wid], sidx_all)
    pltpu.sync_copy(dstw.at[wid], didx_all)
    plsc.subcore_barrier()

    def group(j, carry):
      base = j * nb
      gd = [pltpu.async_copy(rows_hbm.at[sidx_all.at[base + b]], rows[b],
                             gsem[b]) for b in range(nb)]
      sd = []
      for b in range(nb):
        gd[b].wait()
        sd.append(pltpu.async_copy(rows[b], acc.at[didx_all.at[base + b]],
                                   ssem[b], add=True))
      for d_ in sd:
        d_.wait()
      return carry

    lax.fori_loop(0, c_chunks // nb, group, 0)
    plsc.subcore_barrier()
    pltpu.sync_copy(acc.at[pl.ds(sid * rpt, rpt)],
                    out_hbm.at[cid, pl.ds(sid * rpt, rpt)])

  return pl.kernel(
      body,
      out_type=jax.ShapeDtypeStruct((_NC, npad, d), jnp.float32),
      mesh=mesh,
      scratch_types=[
          pltpu.VMEM((c_chunks, _K), jnp.int32),
          pltpu.VMEM((c_chunks, _K), jnp.int32),
      ] + [pltpu.VMEM((_K, d), jnp.float32)] * nb + [
          pltpu.VMEM_SHARED((npad, d), jnp.float32),
      ] + [pltpu.SemaphoreType.DMA] * (2 * nb),
  )


def _dinv(degp0, degp1):
  return lax.rsqrt(degp0[:, 0:1] + degp1[:, 0:1] + 1.0)


def _scale_body(degp_ref, x_ref, y_ref):
  y_ref[...] = x_ref[...] * _dinv(degp_ref[0], degp_ref[1])


def _dense_body(degp_ref, aggp_ref, y_ref, w1_ref, b1_ref, w2_ref, y2_ref):
  dinv = _dinv(degp_ref[0], degp_ref[1])
  pre = (aggp_ref[0] + aggp_ref[1] + y_ref[...]) * dinv
  h = jnp.dot(pre, w1_ref[...], preferred_element_type=jnp.float32)
  x1 = jnp.maximum(h + b1_ref[...], 0.0)
  h2 = jnp.dot(x1, w2_ref[...], preferred_element_type=jnp.float32)
  y2_ref[...] = h2 * dinv


def _final_body(degp_ref, aggp_ref, y2_ref, b2_ref, out_ref):
  dinv = _dinv(degp_ref[0], degp_ref[1])
  o = (aggp_ref[0] + aggp_ref[1] + y2_ref[...]) * dinv + b2_ref[...]
  m = jnp.max(o, axis=1, keepdims=True)
  s = jnp.sum(jnp.exp(o - m), axis=1, keepdims=True)
  out_ref[...] = (o - m) - jnp.log(s)


def _row_specs(npad, d):
  degp = pl.BlockSpec((2, _R, 128), lambda i: (0, i, 0))
  rows = pl.BlockSpec((_R, d), lambda i: (i, 0))
  aggp = pl.BlockSpec((2, _R, d), lambda i: (0, i, 0))
  return degp, rows, aggp


def kernel(x, edge_index, W1, b1, W2, b2):
  n, d_in = x.shape
  e = edge_index.shape[1]
  hid = W1.shape[1]
  d_out = W2.shape[1]

  npad = ((n + 1 + 255) // 256) * 256
  c_chunks = -(-e // (_W * _K))
  epad = _W * _K * c_chunks
  pad = epad - e

  x = x.astype(jnp.float32)
  src = jnp.concatenate([edge_index[0],
                         jnp.zeros((pad,), edge_index.dtype)])
  sink = n + (jnp.arange(pad, dtype=edge_index.dtype) % (npad - n))
  dst = jnp.concatenate([edge_index[1], sink])
  srcw = src.reshape(_W, c_chunks, _K)
  dstw = dst.reshape(_W, c_chunks, _K)
  x_p = jnp.pad(x, ((0, npad - n), (0, 0)))
  zeros_d = jnp.zeros((npad, d_in), jnp.float32)
  ones_d = jnp.ones((_K, d_in), jnp.float32)

  degp = _deg_kernel(npad, c_chunks, d_in)(dstw, zeros_d, ones_d)

  grid = (npad // _R,)
  degp_s, row_s, aggp_s = _row_specs(npad, d_in)

  y = pl.pallas_call(
      _scale_body,
      grid=grid,
      in_specs=[degp_s, row_s],
      out_specs=row_s,
      out_shape=jax.ShapeDtypeStruct((npad, d_in), jnp.float32),
  )(degp, x_p)

  aggp1 = _agg_kernel(npad, c_chunks, d_in)(y, srcw, dstw, zeros_d)

  y2 = pl.pallas_call(
      _dense_body,
      grid=grid,
      in_specs=[
          degp_s, aggp_s, row_s,
          pl.BlockSpec((d_in, hid), lambda i: (0, 0)),
          pl.BlockSpec((1, hid), lambda i: (0, 0)),
          pl.BlockSpec((hid, d_out), lambda i: (0, 0)),
      ],
      out_specs=pl.BlockSpec((_R, d_out), lambda i: (i, 0)),
      out_shape=jax.ShapeDtypeStruct((npad, d_out), jnp.float32),
  )(degp, aggp1, y, W1, b1.reshape(1, hid), W2)

  aggp2 = _agg_kernel(npad, c_chunks, d_out)(y2, srcw, dstw, zeros_d)

  degp_s2, row_s2, aggp_s2 = _row_specs(npad, d_out)
  out = pl.pallas_call(
      _final_body,
      grid=grid,
      in_specs=[degp_s2, aggp_s2, row_s2,
                pl.BlockSpec((1, d_out), lambda i: (0, 0))],
      out_specs=row_s2,
      out_shape=jax.ShapeDtypeStruct((npad, d_out), jnp.float32),
  )(degp, aggp2, y2, b2.reshape(1, d_out))

  return out[:n]


# speedup vs baseline: 38.5453x; 2.1391x over previous
import functools

import jax
import jax.numpy as jnp
from jax import lax
from jax.experimental import pallas as pl
from jax.experimental.pallas import tpu as pltpu
from jax.experimental.pallas import tpu_sc as plsc

_NC = 2
_NS = 16
_W = _NC * _NS
_K = 128
_R = 512


def _deg_kernel(npad, c_chunks, w=128):
  rpt = npad // _NS
  mesh = plsc.VectorSubcoreMesh(core_axis_name="c", subcore_axis_name="s")

  nb = 2
  assert c_chunks % nb == 0

  def body(dstw, zeros_hbm, ones_hbm, out_hbm, didx_all, ones_v, acc, *sems):
    cid = lax.axis_index("c")
    sid = lax.axis_index("s")
    wid = sid * _NC + cid
    pltpu.sync_copy(zeros_hbm.at[pl.ds(sid * rpt, rpt)],
                    acc.at[pl.ds(sid * rpt, rpt)])
    pltpu.sync_copy(ones_hbm, ones_v)
    pltpu.sync_copy(dstw.at[wid], didx_all)
    plsc.subcore_barrier()

    def group(j, carry):
      base = j * nb
      ds = [pltpu.async_copy(ones_v, acc.at[didx_all.at[base + b]], sems[b],
                             add=True) for b in range(nb)]
      for d_ in ds:
        d_.wait()
      return carry

    lax.fori_loop(0, c_chunks // nb, group, 0)
    plsc.subcore_barrier()
    pltpu.sync_copy(acc.at[pl.ds(sid * rpt, rpt)],
                    out_hbm.at[cid, pl.ds(sid * rpt, rpt)])

  return pl.kernel(
      body,
      out_type=jax.ShapeDtypeStruct((_NC, npad, w), jnp.float32),
      mesh=mesh,
      scratch_types=[
          pltpu.VMEM((c_chunks, _K), jnp.int32),
          pltpu.VMEM((_K, w), jnp.float32),
          pltpu.VMEM_SHARED((npad, w), jnp.float32),
      ] + [pltpu.SemaphoreType.DMA] * nb,
  )


def _agg_kernel(npad, c_chunks, d):
  rpt = npad // _NS
  mesh = plsc.VectorSubcoreMesh(core_axis_name="c", subcore_axis_name="s")

  nb = 2
  assert c_chunks % nb == 0

  def body(rows_hbm, srcw, dstw, zeros_hbm, out_hbm, sidx_all, didx_all,
           *bufs_and_sems):
    rows = bufs_and_sems[:nb]
    acc = bufs_and_sems[nb]
    gsem = bufs_and_sems[nb + 1:2 * nb + 1]
    ssem = bufs_and_sems[2 * nb + 1:]
    cid = lax.axis_index("c")
    sid = lax.axis_index("s")
    wid = sid * _NC + cid
    pltpu.sync_copy(zeros_hbm.at[pl.ds(sid * rpt, rpt)],
                    acc.at[pl.ds(sid * rpt, rpt)])
    pltpu.sync_copy(srcw.at[wid], sidx_all)
    pltpu.sync_copy(dstw.at[wid], didx_all)
    plsc.subcore_barrier()

    def group(j, carry):
      base = j * nb
      gd = [pltpu.async_copy(rows_hbm.at[sidx_all.at[base + b]], rows[b],
                             gsem[b]) for b in range(nb)]
      sd = []
      for b in range(nb):
        gd[b].wait()
        sd.append(pltpu.async_copy(rows[b], acc.at[didx_all.at[base + b]],
                                   ssem[b], add=True))
      for d_ in sd:
        d_.wait()
      return carry

    lax.fori_loop(0, c_chunks // nb, group, 0)
    plsc.subcore_barrier()
    pltpu.sync_copy(acc.at[pl.ds(sid * rpt, rpt)],
                    out_hbm.at[cid, pl.ds(sid * rpt, rpt)])

  return pl.kernel(
      body,
      out_type=jax.ShapeDtypeStruct((_NC, npad, d), jnp.float32),
      mesh=mesh,
      scratch_types=[
          pltpu.VMEM((c_chunks, _K), jnp.int32),
          pltpu.VMEM((c_chunks, _K), jnp.int32),
      ] + [pltpu.VMEM((_K, d), jnp.float32)] * nb + [
          pltpu.VMEM_SHARED((npad, d), jnp.float32),
      ] + [pltpu.SemaphoreType.DMA] * (2 * nb),
  )


def _dinv(degp0, degp1):
  return lax.rsqrt(degp0[:, 0:1] + degp1[:, 0:1] + 1.0)


def _scale_body(degp_ref, x_ref, y_ref):
  y_ref[...] = x_ref[...] * _dinv(degp_ref[0], degp_ref[1])


def _dense_body(degp_ref, aggp_ref, y_ref, w1_ref, b1_ref, w2_ref, y2_ref):
  dinv = _dinv(degp_ref[0], degp_ref[1])
  pre = (aggp_ref[0] + aggp_ref[1] + y_ref[...]) * dinv
  h = jnp.dot(pre, w1_ref[...], preferred_element_type=jnp.float32)
  x1 = jnp.maximum(h + b1_ref[...], 0.0)
  h2 = jnp.dot(x1, w2_ref[...], preferred_element_type=jnp.float32)
  y2_ref[...] = h2 * dinv


def _final_body(degp_ref, aggp_ref, y2_ref, b2_ref, out_ref):
  dinv = _dinv(degp_ref[0], degp_ref[1])
  o = (aggp_ref[0] + aggp_ref[1] + y2_ref[...]) * dinv + b2_ref[...]
  m = jnp.max(o, axis=1, keepdims=True)
  s = jnp.sum(jnp.exp(o - m), axis=1, keepdims=True)
  out_ref[...] = (o - m) - jnp.log(s)


def _row_specs(npad, d):
  degp = pl.BlockSpec((2, _R, 128), lambda i: (0, i, 0))
  rows = pl.BlockSpec((_R, d), lambda i: (i, 0))
  aggp = pl.BlockSpec((2, _R, d), lambda i: (0, i, 0))
  return degp, rows, aggp


def kernel(x, edge_index, W1, b1, W2, b2):
  n, d_in = x.shape
  e = edge_index.shape[1]
  hid = W1.shape[1]
  d_out = W2.shape[1]

  npad = ((n + 1 + 255) // 256) * 256
  c_chunks = -(-e // (_W * _K))
  epad = _W * _K * c_chunks
  pad = epad - e

  x = x.astype(jnp.float32)
  src = jnp.concatenate([edge_index[0],
                         jnp.arange(pad, dtype=edge_index.dtype) % n])
  sink = n + (jnp.arange(pad, dtype=edge_index.dtype) % (npad - n))
  dst = jnp.concatenate([edge_index[1], sink])
  srcw = src.reshape(_W, c_chunks, _K)
  dstw = dst.reshape(_W, c_chunks, _K)
  x_p = jnp.pad(x, ((0, npad - n), (0, 0)))
  zeros_d = jnp.zeros((npad, d_in), jnp.float32)
  ones_d = jnp.ones((_K, d_in), jnp.float32)

  degp = _deg_kernel(npad, c_chunks, d_in)(dstw, zeros_d, ones_d)

  grid = (npad // _R,)
  degp_s, row_s, aggp_s = _row_specs(npad, d_in)

  y = pl.pallas_call(
      _scale_body,
      grid=grid,
      in_specs=[degp_s, row_s],
      out_specs=row_s,
      out_shape=jax.ShapeDtypeStruct((npad, d_in), jnp.float32),
  )(degp, x_p)

  aggp1 = _agg_kernel(npad, c_chunks, d_in)(y, srcw, dstw, zeros_d)

  y2 = pl.pallas_call(
      _dense_body,
      grid=grid,
      in_specs=[
          degp_s, aggp_s, row_s,
          pl.BlockSpec((d_in, hid), lambda i: (0, 0)),
          pl.BlockSpec((1, hid), lambda i: (0, 0)),
          pl.BlockSpec((hid, d_out), lambda i: (0, 0)),
      ],
      out_specs=pl.BlockSpec((_R, d_out), lambda i: (i, 0)),
      out_shape=jax.ShapeDtypeStruct((npad, d_out), jnp.float32),
  )(degp, aggp1, y, W1, b1.reshape(1, hid), W2)

  aggp2 = _agg_kernel(npad, c_chunks, d_out)(y2, srcw, dstw, zeros_d)

  degp_s2, row_s2, aggp_s2 = _row_specs(npad, d_out)
  out = pl.pallas_call(
      _final_body,
      grid=grid,
      in_specs=[degp_s2, aggp_s2, row_s2,
                pl.BlockSpec((1, d_out), lambda i: (0, 0))],
      out_specs=row_s2,
      out_shape=jax.ShapeDtypeStruct((npad, d_out), jnp.float32),
  )(degp, aggp2, y2, b2.reshape(1, d_out))

  return out[:n]
